# baseline (speedup 1.0000x reference)
import functools

import jax
import jax.numpy as jnp
from jax import lax
from jax.experimental import pallas as pl
from jax.experimental.pallas import tpu as pltpu
from jax.experimental.pallas import tpu_sc as plsc

N_NODES = 10000
D_FEAT = 128
N_EDGES = 320000

NC = 2
NS = 16
HALF = N_NODES // NC
EPT = N_EDGES // NS
NVEC = EPT // 16
SEL = EPT + 256
ACC_R = 5120
TRASH = HALF + 8
K = 128
DR = 312

_mesh = plsc.VectorSubcoreMesh(core_axis_name="c", subcore_axis_name="s")


@functools.partial(
    pl.kernel,
    mesh=_mesh,
    out_type=jax.ShapeDtypeStruct((N_NODES, D_FEAT), jnp.float32),
    scratch_types=[
        pltpu.VMEM_SHARED((ACC_R, D_FEAT), jnp.float32),
        pltpu.VMEM((SEL,), jnp.int32),
        pltpu.VMEM((SEL,), jnp.int32),
        pltpu.VMEM((K, D_FEAT), jnp.float32),
        pltpu.VMEM((K, D_FEAT), jnp.float32),
        pltpu.VMEM((K,), jnp.int32),
        pltpu.VMEM((K,), jnp.int32),
        pltpu.VMEM((K,), jnp.int32),
        pltpu.VMEM((K,), jnp.int32),
        pltpu.VMEM((D_FEAT,), jnp.float32),
        pltpu.SemaphoreType.DMA,
        pltpu.SemaphoreType.DMA,
        pltpu.SemaphoreType.DMA,
    ],
    compiler_params=pltpu.CompilerParams(needs_layout_passes=False),
)
def _gcn_sc(feat, srcv, dstv, w, out, acc, src_sl, dst_sl, rows0, rows1,
            gsrc0, gdst0, gsrc1, gdst1, wv, sem0, sem1, esem):
    c = lax.axis_index("c")
    s = lax.axis_index("s")
    zero16 = jnp.zeros((16,), jnp.float32)

    e0 = pltpu.async_copy(srcv.at[pl.ds(s * EPT, EPT)],
                          src_sl.at[pl.ds(0, EPT)], esem)
    e1 = pltpu.async_copy(dstv.at[pl.ds(s * EPT, EPT)],
                          dst_sl.at[pl.ds(0, EPT)], esem)
    pltpu.sync_copy(w, wv)

    for r in range(64):
        for j in range(8):
            rows0[r, pl.ds(j * 16, 16)] = zero16
    for kk in range(5):
        pltpu.sync_copy(rows0.at[pl.ds(0, 64)],
                        acc.at[pl.ds(s * 320 + kk * 64, 64)])
    e0.wait()
    e1.wait()

    plsc.subcore_barrier()

    base_node = c * HALF

    def fbody(i, off):
        sl = src_sl[pl.ds(i * 16, 16)]
        dl = dst_sl[pl.ds(i * 16, 16)] - base_node
        m = (dl >= 0) & (dl < HALF)
        mi = m.astype(jnp.int32)
        pos = jnp.cumsum(mi) - 1 + off
        plsc.store_scatter(src_sl, [pos], sl, mask=m)
        plsc.store_scatter(dst_sl, [pos], dl, mask=m)
        return off + jnp.sum(mi)

    n_sel = lax.fori_loop(0, NVEC, fbody, jnp.int32(0), unroll=4)

    lanes = lax.iota(jnp.int32, 16)
    ones16 = jnp.full((16,), True)
    for i in range(16):
        pidx = n_sel + i * 16 + lanes
        plsc.store_scatter(src_sl, [pidx], jnp.zeros((16,), jnp.int32),
                           mask=ones16)
        plsc.store_scatter(dst_sl, [pidx], jnp.full((16,), TRASH, jnp.int32),
                           mask=ones16)

    npairs = jnp.maximum((n_sel + 255) >> 8, 1)

    def fill(b, gs, gd):
        for kk in range(8):
            gs[pl.ds(kk * 16, 16)] = src_sl[pl.ds(b + kk * 16, 16)]
            gd[pl.ds(kk * 16, 16)] = dst_sl[pl.ds(b + kk * 16, 16)]

    fill(0, gsrc0, gdst0)
    pltpu.async_copy(feat.at[gsrc0], rows0, sem0)

    def pbody(p, _):
        b2 = p * 256
        fill(b2 + K, gsrc1, gdst1)
        pltpu.async_copy(feat.at[gsrc1], rows1, sem1)
        pltpu.make_async_copy(feat.at[gsrc0], rows0, sem0).wait()
        pltpu.sync_copy(rows0, acc.at[gdst0], add=True)

        @pl.when(p + 1 < npairs)
        def _():
            fill(b2 + 2 * K, gsrc0, gdst0)
            pltpu.async_copy(feat.at[gsrc0], rows0, sem0)

        pltpu.make_async_copy(feat.at[gsrc1], rows1, sem1).wait()
        pltpu.sync_copy(rows1, acc.at[gdst1], add=True)
        return 0

    lax.fori_loop(0, npairs, pbody, 0)

    plsc.subcore_barrier()

    wregs = [wv[pl.ds(j * 16, 16)] for j in range(8)]

    def scale_rows(nr):
        def mbody(r, _):
            for j in range(8):
                rows0[r, pl.ds(j * 16, 16)] = rows0[r, pl.ds(j * 16, 16)] * wregs[j]
            return 0
        lax.fori_loop(0, nr, mbody, 0, unroll=2)

    r0 = s * DR
    for kk in range(3):
        rs = r0 + kk * 104
        pltpu.sync_copy(acc.at[pl.ds(rs, 104)], rows0.at[pl.ds(0, 104)])
        scale_rows(104)
        pltpu.sync_copy(rows0.at[pl.ds(0, 104)],
                        out.at[pl.ds(c * HALF + rs, 104)])

    @pl.when(s == NS - 1)
    def _tail():
        pltpu.sync_copy(acc.at[pl.ds(NS * DR, 8)], rows0.at[pl.ds(0, 8)])
        scale_rows(8)
        pltpu.sync_copy(rows0.at[pl.ds(0, 8)],
                        out.at[pl.ds(c * HALF + NS * DR, 8)])


def kernel(features, edge_index, W):
    src = edge_index[0]
    dst = edge_index[1]
    return _gcn_sc(features, src, dst, W)


# speedup vs baseline: 1.1606x; 1.1606x over previous
import functools

import jax
import jax.numpy as jnp
from jax import lax
from jax.experimental import pallas as pl
from jax.experimental.pallas import tpu as pltpu
from jax.experimental.pallas import tpu_sc as plsc

N_NODES = 10000
D_FEAT = 128
N_EDGES = 320000

NC = 2
NS = 16
HALF = N_NODES // NC
EPT = N_EDGES // NS
EC = 400
NCH = EPT // EC
NV = EC // 16
ACC_R = 5008
TRASH = HALF
K = 16
DR = 312
SROWS = 624

_mesh = plsc.VectorSubcoreMesh(core_axis_name="c", subcore_axis_name="s")


@functools.partial(
    pl.kernel,
    mesh=_mesh,
    out_type=jax.ShapeDtypeStruct((N_NODES, D_FEAT), jnp.float32),
    scratch_types=[
        pltpu.VMEM_SHARED((ACC_R, D_FEAT), jnp.float32),
        pltpu.VMEM_SHARED((N_NODES, D_FEAT), jnp.float32),
        pltpu.VMEM((512,), jnp.int32),
        pltpu.VMEM((512,), jnp.int32),
        pltpu.VMEM((32, K), jnp.int32),
        pltpu.VMEM((K, D_FEAT), jnp.float32),
        pltpu.VMEM((K, D_FEAT), jnp.float32),
        pltpu.VMEM((D_FEAT,), jnp.float32),
        pltpu.SemaphoreType.DMA,
        pltpu.SemaphoreType.DMA,
        pltpu.SemaphoreType.DMA,
        pltpu.SemaphoreType.DMA,
    ],
    compiler_params=pltpu.CompilerParams(needs_layout_passes=False),
)
def _gcn_sc(feat, srcv, dstv, w, out, acc, ftab, sstage, dstage, gdst,
            rows0, rows1, wv, sem0, sem1, esem, fsem):
    c = lax.axis_index("c")
    s = lax.axis_index("s")
    zero16 = jnp.zeros((16,), jnp.float32)

    ft = pltpu.async_copy(feat.at[pl.ds(s * SROWS, SROWS)],
                          ftab.at[pl.ds(s * SROWS, SROWS)], fsem)
    pltpu.sync_copy(w, wv)

    for r in range(K):
        for j in range(8):
            rows0[r, pl.ds(j * 16, 16)] = zero16
            rows1[r, pl.ds(j * 16, 16)] = zero16
    r0 = s * DR
    for kk in range(19):
        pltpu.sync_copy(rows0, acc.at[pl.ds(r0 + kk * K, K)])
    pltpu.sync_copy(rows0.at[pl.ds(0, 8)], acc.at[pl.ds(r0 + 304, 8)])

    @pl.when(s == NS - 1)
    def _ztail():
        pltpu.sync_copy(rows0.at[pl.ds(0, 16)],
                        acc.at[pl.ds(NS * DR, ACC_R - NS * DR)])

    e0 = pltpu.async_copy(srcv.at[pl.ds(s * EPT, EC)],
                          sstage.at[pl.ds(0, EC)], esem)
    e1 = pltpu.async_copy(dstv.at[pl.ds(s * EPT, EC)],
                          dstage.at[pl.ds(0, EC)], esem)

    ft.wait()

    @pl.when(s == NS - 1)
    def _ftail():
        pltpu.sync_copy(feat.at[pl.ds(NS * SROWS, N_NODES - NS * SROWS)],
                        ftab.at[pl.ds(NS * SROWS, N_NODES - NS * SROWS)])

    e0.wait()
    e1.wait()

    plsc.subcore_barrier()

    base_node = c * HALF
    lanes = lax.iota(jnp.int32, 16)
    ones16 = jnp.full((16,), True)
    trash16 = jnp.full((16,), TRASH, jnp.int32)
    zeros16i = jnp.zeros((16,), jnp.int32)

    def chunk_body(ch, _):
        def fb(i, off):
            sl = sstage[pl.ds(i * 16, 16)]
            dl = dstage[pl.ds(i * 16, 16)] - base_node
            m = (dl >= 0) & (dl < HALF)
            mi = m.astype(jnp.int32)
            pos = off + jnp.cumsum(mi) - 1
            plsc.store_scatter(sstage, [pos], sl, mask=m)
            plsc.store_scatter(gdst, [pos >> 4, pos & 15], dl, mask=m)
            return off + jnp.sum(mi)

        cnt = lax.fori_loop(0, NV, fb, jnp.int32(0), unroll=5)

        @pl.when(ch + 1 < NCH)
        def _():
            eb = s * EPT + (ch + 1) * EC
            pltpu.async_copy(srcv.at[pl.ds(eb, EC)],
                             sstage.at[pl.ds(0, EC)], esem)
            pltpu.async_copy(dstv.at[pl.ds(eb, EC)],
                             dstage.at[pl.ds(0, EC)], esem)

        for i in range(2):
            pidx = cnt + i * 16 + lanes
            plsc.store_scatter(sstage, [pidx], zeros16i, mask=ones16)
            plsc.store_scatter(gdst, [pidx >> 4, pidx & 15], trash16,
                               mask=ones16)

        nw = (cnt + 15) >> 4
        npr = nw >> 1

        def pair(p, _):
            g0 = pltpu.async_copy(ftab.at[sstage.at[pl.ds(2 * p * K, K)]],
                                  rows0, sem0)
            g1 = pltpu.async_copy(ftab.at[sstage.at[pl.ds((2 * p + 1) * K, K)]],
                                  rows1, sem1)
            g0.wait()
            pltpu.sync_copy(rows0, acc.at[gdst.at[2 * p]], add=True)
            g1.wait()
            pltpu.sync_copy(rows1, acc.at[gdst.at[2 * p + 1]], add=True)
            return 0

        lax.fori_loop(0, npr, pair, 0)

        @pl.when((nw & 1) == 1)
        def _odd():
            g = pltpu.async_copy(ftab.at[sstage.at[pl.ds(2 * npr * K, K)]],
                                 rows0, sem0)
            g.wait()
            pltpu.sync_copy(rows0, acc.at[gdst.at[2 * npr]], add=True)

        @pl.when(ch + 1 < NCH)
        def _w():
            pltpu.make_async_copy(srcv.at[pl.ds(0, EC)],
                                  sstage.at[pl.ds(0, EC)], esem).wait()
            pltpu.make_async_copy(dstv.at[pl.ds(0, EC)],
                                  dstage.at[pl.ds(0, EC)], esem).wait()

        return 0

    lax.fori_loop(0, NCH, chunk_body, 0)

    plsc.subcore_barrier()

    wregs = [wv[pl.ds(j * 16, 16)] for j in range(8)]

    def scale_rows(nr):
        def mbody(r, _):
            for j in range(8):
                rows0[r, pl.ds(j * 16, 16)] = rows0[r, pl.ds(j * 16, 16)] * wregs[j]
            return 0
        lax.fori_loop(0, nr, mbody, 0, unroll=2)

    for kk in range(19):
        rs = r0 + kk * K
        pltpu.sync_copy(acc.at[pl.ds(rs, K)], rows0)
        scale_rows(K)
        pltpu.sync_copy(rows0, out.at[pl.ds(c * HALF + rs, K)])
    pltpu.sync_copy(acc.at[pl.ds(r0 + 304, 8)], rows0.at[pl.ds(0, 8)])
    scale_rows(8)
    pltpu.sync_copy(rows0.at[pl.ds(0, 8)],
                    out.at[pl.ds(c * HALF + r0 + 304, 8)])

    @pl.when(s == NS - 1)
    def _tail():
        pltpu.sync_copy(acc.at[pl.ds(NS * DR, 8)], rows0.at[pl.ds(0, 8)])
        scale_rows(8)
        pltpu.sync_copy(rows0.at[pl.ds(0, 8)],
                        out.at[pl.ds(c * HALF + NS * DR, 8)])


def kernel(features, edge_index, W):
    src = edge_index[0]
    dst = edge_index[1]
    return _gcn_sc(features, src, dst, W)
